# baseline (speedup 1.0000x reference)
import functools

import jax
import jax.numpy as jnp
from jax import lax
from jax.experimental import pallas as pl
from jax.experimental.pallas import tpu as pltpu
from jax.experimental.pallas import tpu_sc as plsc

_NC = 2
_NS = 16
_NW = _NC * _NS

_BLK = 512


def _dist_argmin_body(x_ref, w_ref, idx_ref):
    x = x_ref[...]
    w = w_ref[...]
    mm = lax.dot_general(x, w, (((1,), (1,)), ((), ())),
                         preferred_element_type=jnp.float32)
    x2 = jnp.sum(x * x, axis=1, keepdims=True)
    w2 = jnp.sum(w * w, axis=1)
    d = (x2 + w2[None, :]) - 2.0 * mm
    m = jnp.min(d, axis=1, keepdims=True)
    cols = lax.broadcasted_iota(jnp.int32, d.shape, 1)
    ii = jnp.where(d == m, cols, jnp.int32(1 << 30))
    idx_ref[...] = jnp.min(ii, axis=1, keepdims=True)


def _dist_argmin(x, w):
    rows, _ = x.shape
    k, dd = w.shape
    grid = rows // _BLK
    return pl.pallas_call(
        _dist_argmin_body,
        grid=(grid,),
        in_specs=[
            pl.BlockSpec((_BLK, dd), lambda i: (i, 0)),
            pl.BlockSpec((k, dd), lambda i: (0, 0)),
        ],
        out_specs=pl.BlockSpec((_BLK, 1), lambda i: (i, 0)),
        out_shape=jax.ShapeDtypeStruct((rows, 1), jnp.int32),
    )(x, w)


def _sc_gather_body(w_hbm, idx_hbm, zeros_hbm, ones_hbm, q_hbm, cnt_hbm,
                    idx_v, rows_v, ones_v, shared, sem):
    cid = lax.axis_index("c")
    sid = lax.axis_index("s")
    wid = sid * _NC + cid
    b = idx_v.shape[0]
    base = wid * b
    pltpu.sync_copy(idx_hbm.at[pl.ds(base, b)], idx_v)
    pltpu.sync_copy(ones_hbm, ones_v)
    pltpu.async_copy(w_hbm.at[idx_v], rows_v, sem).wait()
    pltpu.sync_copy(rows_v, q_hbm.at[pl.ds(base, b)])

    @pl.when(sid == 0)
    def _():
        pltpu.sync_copy(zeros_hbm, shared)

    plsc.subcore_barrier()
    pltpu.sync_copy(ones_v, shared.at[idx_v], add=True)
    plsc.subcore_barrier()

    @pl.when(sid == 0)
    def _():
        pltpu.sync_copy(shared, cnt_hbm.at[cid])


def _sc_gather_counts(w, idx, zeros, ones):
    rows = idx.shape[0]
    k, d = w.shape
    b = rows // _NW
    mesh = plsc.VectorSubcoreMesh(core_axis_name="c", subcore_axis_name="s")
    fn = functools.partial(
        pl.kernel,
        mesh=mesh,
        compiler_params=pltpu.CompilerParams(use_tc_tiling_on_sc=False),
        out_type=[
            jax.ShapeDtypeStruct((rows, d), jnp.float32),
            jax.ShapeDtypeStruct((_NC, k, 16), jnp.float32),
        ],
        scratch_types=[
            pltpu.VMEM((b,), jnp.int32),
            pltpu.VMEM((b, d), jnp.float32),
            pltpu.VMEM((b, 16), jnp.float32),
            pltpu.VMEM_SHARED((k, 16), jnp.float32),
            pltpu.SemaphoreType.DMA,
        ],
    )(_sc_gather_body)
    return fn(w, idx, zeros, ones)


def _finalize_body(x_ref, q_ref, cnt_ref, qo_ref, loss_ref, perp_ref, acc_ref):
    i = pl.program_id(0)
    n = pl.num_programs(0)

    @pl.when(i == 0)
    def _():
        acc_ref[...] = jnp.zeros_like(acc_ref)

    x = x_ref[...]
    q = q_ref[...]
    qo_ref[...] = x + (q - x)
    e = x - q
    acc_ref[...] += jnp.sum(e * e, keepdims=True)

    @pl.when(i == n - 1)
    def _():
        rows_total = n * x_ref.shape[0]
        denom = rows_total * x_ref.shape[1]
        loss_ref[...] = acc_ref[...] * (0.25 / denom)
        c = cnt_ref[0] + cnt_ref[1]
        p = c * (1.0 / rows_total)
        s = jnp.sum(p * jnp.log(p + 1e-10), keepdims=True)
        perp_ref[...] = jnp.exp(-s * (1.0 / 16.0))


def _finalize(x, q, cnt):
    rows, d = x.shape
    k = cnt.shape[1]
    grid = rows // _BLK
    return pl.pallas_call(
        _finalize_body,
        grid=(grid,),
        in_specs=[
            pl.BlockSpec((_BLK, d), lambda i: (i, 0)),
            pl.BlockSpec((_BLK, d), lambda i: (i, 0)),
            pl.BlockSpec((_NC, k, 16), lambda i: (0, 0, 0)),
        ],
        out_specs=[
            pl.BlockSpec((_BLK, d), lambda i: (i, 0)),
            pl.BlockSpec((1, 1), lambda i: (0, 0)),
            pl.BlockSpec((1, 1), lambda i: (0, 0)),
        ],
        out_shape=[
            jax.ShapeDtypeStruct((rows, d), jnp.float32),
            jax.ShapeDtypeStruct((1, 1), jnp.float32),
            jax.ShapeDtypeStruct((1, 1), jnp.float32),
        ],
        scratch_shapes=[pltpu.VMEM((1, 1), jnp.float32)],
    )(x, q, cnt)


def kernel(latents, W):
    orig_shape = latents.shape
    d = orig_shape[-1]
    x = latents.reshape(-1, d)
    rows = x.shape[0]

    idx2 = _dist_argmin(x, W)
    idx = idx2.reshape(rows)

    zeros = jnp.zeros((W.shape[0], 16), jnp.float32)
    ones = jnp.ones((rows // _NW, 16), jnp.float32)
    q, cnt = _sc_gather_counts(W, idx, zeros, ones)

    qo, loss, perp = _finalize(x, q, cnt)
    return (qo.reshape(orig_shape), loss.reshape(()), perp.reshape(()))


# speedup vs baseline: 1.2871x; 1.2871x over previous
import functools

import jax
import jax.numpy as jnp
from jax import lax
from jax.experimental import pallas as pl
from jax.experimental.pallas import tpu as pltpu
from jax.experimental.pallas import tpu_sc as plsc

_NC = 2
_NS = 16
_NW = _NC * _NS

_BLK = 512


def _stage_a_body(x_ref, w_ref, idx_ref, loss_ref, perp_ref, acc_ref, cnt_ref):
    i = pl.program_id(0)
    n = pl.num_programs(0)

    @pl.when(i == 0)
    def _():
        acc_ref[...] = jnp.zeros_like(acc_ref)
        cnt_ref[...] = jnp.zeros_like(cnt_ref)

    x = x_ref[...]
    w = w_ref[...]
    mm = lax.dot_general(x, w, (((1,), (1,)), ((), ())),
                         preferred_element_type=jnp.float32)
    x2 = jnp.sum(x * x, axis=1, keepdims=True)
    w2 = jnp.sum(w * w, axis=1)
    d = (x2 + w2[None, :]) - 2.0 * mm
    m = jnp.min(d, axis=1, keepdims=True)
    cols = lax.broadcasted_iota(jnp.int32, d.shape, 1)
    ii = jnp.where(d == m, cols, jnp.int32(1 << 30))
    idx = jnp.min(ii, axis=1, keepdims=True)
    idx_ref[...] = idx

    acc_ref[...] += jnp.sum(m, keepdims=True)
    onehot = jnp.where(cols == idx, 1.0, 0.0).astype(jnp.float32)
    ones = jnp.ones((8, x.shape[0]), jnp.float32)
    cnt_ref[...] += lax.dot_general(ones, onehot, (((1,), (0,)), ((), ())),
                                    preferred_element_type=jnp.float32)

    @pl.when(i == n - 1)
    def _():
        rows_total = n * x_ref.shape[0]
        denom = rows_total * x_ref.shape[1]
        loss_ref[...] = acc_ref[...] * (0.25 / denom)
        p = cnt_ref[0:1, :] * (1.0 / rows_total)
        s = jnp.sum(p * jnp.log(p + 1e-10), keepdims=True)
        perp_ref[...] = jnp.exp(-s)


def _stage_a(x, w):
    rows, dd = x.shape
    k = w.shape[0]
    grid = rows // _BLK
    return pl.pallas_call(
        _stage_a_body,
        grid=(grid,),
        in_specs=[
            pl.BlockSpec((_BLK, dd), lambda i: (i, 0)),
            pl.BlockSpec((k, dd), lambda i: (0, 0)),
        ],
        out_specs=[
            pl.BlockSpec((_BLK, 1), lambda i: (i, 0)),
            pl.BlockSpec((1, 1), lambda i: (0, 0)),
            pl.BlockSpec((1, 1), lambda i: (0, 0)),
        ],
        out_shape=[
            jax.ShapeDtypeStruct((rows, 1), jnp.int32),
            jax.ShapeDtypeStruct((1, 1), jnp.float32),
            jax.ShapeDtypeStruct((1, 1), jnp.float32),
        ],
        scratch_shapes=[
            pltpu.VMEM((1, 1), jnp.float32),
            pltpu.VMEM((8, k), jnp.float32),
        ],
    )(x, w)


def _sc_gather_body(w_hbm, idx_hbm, q_hbm, idx_v, rows_v, sem):
    cid = lax.axis_index("c")
    sid = lax.axis_index("s")
    wid = sid * _NC + cid
    b = idx_v.shape[0]
    base = wid * b
    pltpu.sync_copy(idx_hbm.at[pl.ds(base, b)], idx_v)
    pltpu.async_copy(w_hbm.at[idx_v], rows_v, sem).wait()
    pltpu.sync_copy(rows_v, q_hbm.at[pl.ds(base, b)])


def _sc_gather(w, idx):
    rows = idx.shape[0]
    k, d = w.shape
    b = rows // _NW
    mesh = plsc.VectorSubcoreMesh(core_axis_name="c", subcore_axis_name="s")
    fn = functools.partial(
        pl.kernel,
        mesh=mesh,
        compiler_params=pltpu.CompilerParams(use_tc_tiling_on_sc=False),
        out_type=jax.ShapeDtypeStruct((rows, d), jnp.float32),
        scratch_types=[
            pltpu.VMEM((b,), jnp.int32),
            pltpu.VMEM((b, d), jnp.float32),
            pltpu.SemaphoreType.DMA,
        ],
    )(_sc_gather_body)
    return fn(w, idx)


def kernel(latents, W):
    orig_shape = latents.shape
    d = orig_shape[-1]
    x = latents.reshape(-1, d)
    rows = x.shape[0]

    idx2, loss, perp = _stage_a(x, W)
    q = _sc_gather(W, idx2.reshape(rows))
    return (q.reshape(orig_shape), loss.reshape(()), perp.reshape(()))


# speedup vs baseline: 1.3225x; 1.0275x over previous
import functools

import jax
import jax.numpy as jnp
from jax import lax
from jax.experimental import pallas as pl
from jax.experimental.pallas import tpu as pltpu
from jax.experimental.pallas import tpu_sc as plsc

_NC = 2
_NS = 16
_NW = _NC * _NS

_BLK = 512


def _stage_a_body(x_ref, w_ref, idx_ref, loss_ref, perp_ref, acc_ref, cnt_ref):
    i = pl.program_id(0)
    n = pl.num_programs(0)

    @pl.when(i == 0)
    def _():
        acc_ref[...] = jnp.zeros_like(acc_ref)
        cnt_ref[...] = jnp.zeros_like(cnt_ref)

    x = x_ref[...]
    w = w_ref[...]
    mm = lax.dot_general(x, w, (((1,), (1,)), ((), ())),
                         preferred_element_type=jnp.float32)
    x2 = jnp.sum(x * x, axis=1, keepdims=True)
    w2 = jnp.sum(w * w, axis=1)
    d = (x2 + w2[None, :]) - 2.0 * mm
    m = jnp.min(d, axis=1, keepdims=True)
    cols = lax.broadcasted_iota(jnp.int32, d.shape, 1)
    ii = jnp.where(d == m, cols, jnp.int32(1 << 30))
    idx = jnp.min(ii, axis=1, keepdims=True)
    idx_ref[...] = idx.reshape(idx_ref.shape)

    acc_ref[...] += jnp.sum(m, keepdims=True)
    onehot = jnp.where(cols == idx, 1.0, 0.0).astype(jnp.float32)
    ones = jnp.ones((8, x.shape[0]), jnp.float32)
    cnt_ref[...] += lax.dot_general(ones, onehot, (((1,), (0,)), ((), ())),
                                    preferred_element_type=jnp.float32)

    @pl.when(i == n - 1)
    def _():
        rows_total = n * x_ref.shape[0]
        denom = rows_total * x_ref.shape[1]
        loss_ref[...] = acc_ref[...] * (0.25 / denom)
        p = cnt_ref[0:1, :] * (1.0 / rows_total)
        s = jnp.sum(p * jnp.log(p + 1e-10), keepdims=True)
        perp_ref[...] = jnp.exp(-s)


def _stage_a(x, w):
    rows, dd = x.shape
    k = w.shape[0]
    grid = rows // _BLK
    return pl.pallas_call(
        _stage_a_body,
        grid=(grid,),
        in_specs=[
            pl.BlockSpec((_BLK, dd), lambda i: (i, 0)),
            pl.BlockSpec((k, dd), lambda i: (0, 0)),
        ],
        out_specs=[
            pl.BlockSpec((_BLK,), lambda i: (i,)),
            pl.BlockSpec((1, 1), lambda i: (0, 0)),
            pl.BlockSpec((1, 1), lambda i: (0, 0)),
        ],
        out_shape=[
            jax.ShapeDtypeStruct((rows,), jnp.int32),
            jax.ShapeDtypeStruct((1, 1), jnp.float32),
            jax.ShapeDtypeStruct((1, 1), jnp.float32),
        ],
        scratch_shapes=[
            pltpu.VMEM((1, 1), jnp.float32),
            pltpu.VMEM((8, k), jnp.float32),
        ],
    )(x, w)


def _sc_gather_body(w_hbm, idx_hbm, q_hbm, idx_v, rows_v, sem):
    cid = lax.axis_index("c")
    sid = lax.axis_index("s")
    wid = sid * _NC + cid
    b = idx_v.shape[0]
    per_batch = q_hbm.shape[1] // b
    bi = wid // per_batch
    off = (wid % per_batch) * b
    pltpu.sync_copy(idx_hbm.at[pl.ds(wid * b, b)], idx_v)
    pltpu.async_copy(w_hbm.at[idx_v], rows_v, sem).wait()
    pltpu.sync_copy(rows_v, q_hbm.at[bi, pl.ds(off, b)])


def _sc_gather(w, idx, out_shape):
    rows = idx.shape[0]
    k, d = w.shape
    b = rows // _NW
    mesh = plsc.VectorSubcoreMesh(core_axis_name="c", subcore_axis_name="s")
    fn = functools.partial(
        pl.kernel,
        mesh=mesh,
        compiler_params=pltpu.CompilerParams(use_tc_tiling_on_sc=False),
        out_type=jax.ShapeDtypeStruct(out_shape, jnp.float32),
        scratch_types=[
            pltpu.VMEM((b,), jnp.int32),
            pltpu.VMEM((b, d), jnp.float32),
            pltpu.SemaphoreType.DMA,
        ],
    )(_sc_gather_body)
    return fn(w, idx)


def kernel(latents, W):
    orig_shape = latents.shape
    d = orig_shape[-1]
    x = latents.reshape(-1, d)
    rows = x.shape[0]

    idx, loss, perp = _stage_a(x, W)
    q = _sc_gather(W, idx, orig_shape)
    return (q, loss.reshape(()), perp.reshape(()))


# speedup vs baseline: 1.3947x; 1.0546x over previous
import functools

import jax
import jax.numpy as jnp
from jax import lax
from jax.experimental import pallas as pl
from jax.experimental.pallas import tpu as pltpu
from jax.experimental.pallas import tpu_sc as plsc

_NC = 2
_NS = 16
_NW = _NC * _NS

_BLK = 512


def _stage_a_body(x_ref, w_ref, idx_ref, loss_ref, perp_ref, acc_ref, cnt_ref,
                  nw_ref, w2_ref):
    i = pl.program_id(0)
    n = pl.num_programs(0)
    blk = x_ref.shape[0]
    k = w_ref.shape[0]
    nch = k // 128

    @pl.when(i == 0)
    def _():
        acc_ref[...] = jnp.zeros_like(acc_ref)
        cnt_ref[...] = jnp.zeros_like(cnt_ref)
        w = w_ref[...]
        nw_ref[...] = -(w + w)
        o8 = jnp.ones((8, w.shape[1]), jnp.float32)
        w2_ref[...] = lax.dot_general(w * w, o8, (((1,), (1,)), ((), ())),
                                      preferred_element_type=jnp.float32)

    x = x_ref[...]
    ones8 = jnp.ones((8, x.shape[1]), jnp.float32)
    x2row = lax.dot_general(ones8, x * x, (((1,), (1,)), ((), ())),
                            preferred_element_type=jnp.float32)[0:1]
    w2 = w2_ref[:, 0:1]
    ji = lax.broadcasted_iota(jnp.int32, (128, blk), 0).astype(jnp.float32)

    minval = None
    minidx = None
    for kb in range(nch):
        nwk = nw_ref[kb * 128:(kb + 1) * 128, :]
        mm2k = lax.dot_general(nwk, x, (((1,), (1,)), ((), ())),
                               preferred_element_type=jnp.float32)
        dk = (x2row + w2[kb * 128:(kb + 1) * 128]) + mm2k
        if kb == 0:
            minval = dk
            minidx = ji
        else:
            better = dk < minval
            minval = jnp.where(better, dk, minval)
            minidx = jnp.where(better, ji + float(kb * 128), minidx)

    m = jnp.min(minval, axis=0, keepdims=True)
    cand = jnp.where(minval == m, minidx, jnp.float32(2.0 ** 30))
    idxf = jnp.min(cand, axis=0, keepdims=True)
    idx_ref[...] = idxf.astype(jnp.int32).reshape(idx_ref.shape)

    acc_ref[...] += jnp.sum(m, keepdims=True)
    rowsc = lax.broadcasted_iota(jnp.int32, (k, blk), 0).astype(jnp.float32)
    onehot = jnp.where(rowsc == idxf, 1.0, 0.0).astype(jnp.float32)
    ones = jnp.ones((8, blk), jnp.float32)
    cnt_ref[...] += lax.dot_general(ones, onehot, (((1,), (1,)), ((), ())),
                                    preferred_element_type=jnp.float32)

    @pl.when(i == n - 1)
    def _():
        rows_total = n * x_ref.shape[0]
        denom = rows_total * x_ref.shape[1]
        loss_ref[...] = acc_ref[...] * (0.25 / denom)
        p = cnt_ref[0:1, :] * (1.0 / rows_total)
        s = jnp.sum(p * jnp.log(p + 1e-10), keepdims=True)
        perp_ref[...] = jnp.exp(-s)


def _stage_a(x, w):
    rows, dd = x.shape
    k = w.shape[0]
    grid = rows // _BLK
    return pl.pallas_call(
        _stage_a_body,
        grid=(grid,),
        in_specs=[
            pl.BlockSpec((_BLK, dd), lambda i: (i, 0)),
            pl.BlockSpec((k, dd), lambda i: (0, 0)),
        ],
        out_specs=[
            pl.BlockSpec((_BLK,), lambda i: (i,)),
            pl.BlockSpec((1, 1), lambda i: (0, 0)),
            pl.BlockSpec((1, 1), lambda i: (0, 0)),
        ],
        out_shape=[
            jax.ShapeDtypeStruct((rows,), jnp.int32),
            jax.ShapeDtypeStruct((1, 1), jnp.float32),
            jax.ShapeDtypeStruct((1, 1), jnp.float32),
        ],
        scratch_shapes=[
            pltpu.VMEM((1, 1), jnp.float32),
            pltpu.VMEM((8, k), jnp.float32),
            pltpu.VMEM((k, dd), jnp.float32),
            pltpu.VMEM((k, 8), jnp.float32),
        ],
    )(x, w)


def _sc_gather_body(w_hbm, idx_hbm, q_hbm, idx_v, rows_v, sem):
    cid = lax.axis_index("c")
    sid = lax.axis_index("s")
    wid = sid * _NC + cid
    b = idx_v.shape[0]
    per_batch = q_hbm.shape[1] // b
    bi = wid // per_batch
    off = (wid % per_batch) * b
    pltpu.sync_copy(idx_hbm.at[pl.ds(wid * b, b)], idx_v)
    pltpu.async_copy(w_hbm.at[idx_v], rows_v, sem).wait()
    pltpu.sync_copy(rows_v, q_hbm.at[bi, pl.ds(off, b)])


def _sc_gather(w, idx, out_shape):
    rows = idx.shape[0]
    k, d = w.shape
    b = rows // _NW
    mesh = plsc.VectorSubcoreMesh(core_axis_name="c", subcore_axis_name="s")
    fn = functools.partial(
        pl.kernel,
        mesh=mesh,
        compiler_params=pltpu.CompilerParams(use_tc_tiling_on_sc=False),
        out_type=jax.ShapeDtypeStruct(out_shape, jnp.float32),
        scratch_types=[
            pltpu.VMEM((b,), jnp.int32),
            pltpu.VMEM((b, d), jnp.float32),
            pltpu.SemaphoreType.DMA,
        ],
    )(_sc_gather_body)
    return fn(w, idx)


def kernel(latents, W):
    orig_shape = latents.shape
    d = orig_shape[-1]
    x = latents.reshape(-1, d)
    rows = x.shape[0]

    idx, loss, perp = _stage_a(x, W)
    q = _sc_gather(W, idx, orig_shape)
    return (q, loss.reshape(()), perp.reshape(()))


# speedup vs baseline: 1.4258x; 1.0223x over previous
import functools

import jax
import jax.numpy as jnp
from jax import lax
from jax.experimental import pallas as pl
from jax.experimental.pallas import tpu as pltpu
from jax.experimental.pallas import tpu_sc as plsc

_NC = 2
_NS = 16
_NW = _NC * _NS

_BLK = 512


def _stage_a_body(x_ref, w_ref, idx_ref, loss_ref, perp_ref, acc_ref, cnt_ref,
                  nw_ref, w2_ref):
    i = pl.program_id(0)
    n = pl.num_programs(0)
    blk = x_ref.shape[0]
    k = w_ref.shape[0]
    nch = k // 128

    @pl.when(i == 0)
    def _():
        acc_ref[...] = jnp.zeros_like(acc_ref)
        cnt_ref[...] = jnp.zeros_like(cnt_ref)
        w = w_ref[...]
        nw_ref[...] = -(w + w)
        o8 = jnp.ones((8, w.shape[1]), jnp.float32)
        w2_ref[...] = lax.dot_general(w * w, o8, (((1,), (1,)), ((), ())),
                                      preferred_element_type=jnp.float32)

    x = x_ref[...]
    ones8 = jnp.ones((8, x.shape[1]), jnp.float32)
    x2row = lax.dot_general(ones8, x * x, (((1,), (1,)), ((), ())),
                            preferred_element_type=jnp.float32)[0:1]
    w2 = w2_ref[:, 0:1]
    ji = lax.broadcasted_iota(jnp.int32, (128, blk), 0).astype(jnp.float32)

    minval = None
    minidx = None
    for kb in range(nch):
        nwk = nw_ref[kb * 128:(kb + 1) * 128, :]
        mm2k = lax.dot_general(nwk, x, (((1,), (1,)), ((), ())),
                               preferred_element_type=jnp.float32)
        dk = (x2row + w2[kb * 128:(kb + 1) * 128]) + mm2k
        if kb == 0:
            minval = dk
            minidx = ji
        else:
            better = dk < minval
            minval = jnp.where(better, dk, minval)
            minidx = jnp.where(better, ji + float(kb * 128), minidx)

    m = jnp.min(minval, axis=0, keepdims=True)
    cand = jnp.where(minval == m, minidx, jnp.float32(2.0 ** 30))
    idxf = jnp.min(cand, axis=0, keepdims=True)
    idx_ref[...] = idxf.astype(jnp.int32).reshape(idx_ref.shape)

    acc_ref[...] += jnp.sum(m, keepdims=True)
    rowsc = lax.broadcasted_iota(jnp.int32, (k, blk), 0).astype(jnp.float32)
    onehot = jnp.where(rowsc == idxf, 1.0, 0.0).astype(jnp.float32)
    ones = jnp.ones((8, blk), jnp.float32)
    cnt_ref[...] += lax.dot_general(ones, onehot, (((1,), (1,)), ((), ())),
                                    preferred_element_type=jnp.float32)

    @pl.when(i == n - 1)
    def _():
        rows_total = n * x_ref.shape[0]
        denom = rows_total * x_ref.shape[1]
        loss_ref[...] = acc_ref[...] * (0.25 / denom)
        p = cnt_ref[0:1, :] * (1.0 / rows_total)
        s = jnp.sum(p * jnp.log(p + 1e-10), keepdims=True)
        perp_ref[...] = jnp.exp(-s)


def _stage_a(x, w):
    rows, dd = x.shape
    k = w.shape[0]
    grid = rows // _BLK
    return pl.pallas_call(
        _stage_a_body,
        grid=(grid,),
        in_specs=[
            pl.BlockSpec((_BLK, dd), lambda i: (i, 0)),
            pl.BlockSpec((k, dd), lambda i: (0, 0)),
        ],
        out_specs=[
            pl.BlockSpec((_BLK,), lambda i: (i,)),
            pl.BlockSpec((1, 1), lambda i: (0, 0)),
            pl.BlockSpec((1, 1), lambda i: (0, 0)),
        ],
        out_shape=[
            jax.ShapeDtypeStruct((rows,), jnp.int32),
            jax.ShapeDtypeStruct((1, 1), jnp.float32),
            jax.ShapeDtypeStruct((1, 1), jnp.float32),
        ],
        scratch_shapes=[
            pltpu.VMEM((1, 1), jnp.float32),
            pltpu.VMEM((8, k), jnp.float32),
            pltpu.VMEM((k, dd), jnp.float32),
            pltpu.VMEM((k, 8), jnp.float32),
        ],
    )(x, w)


def _sc_gather_body(w_hbm, idx_hbm, q_hbm, idx_v, rows_v, sem):
    cid = lax.axis_index("c")
    sid = lax.axis_index("s")
    wid = sid * _NC + cid
    b = idx_v.shape[0]
    base = wid * b
    pltpu.sync_copy(idx_hbm.at[pl.ds(base, b)], idx_v)
    pltpu.async_copy(w_hbm.at[idx_v], rows_v, sem).wait()
    pltpu.sync_copy(rows_v, q_hbm.at[pl.ds(base, b)])


def _sc_gather(w_pad, idx):
    rows = idx.shape[0]
    k, dpad = w_pad.shape
    b = rows // _NW
    mesh = plsc.VectorSubcoreMesh(core_axis_name="c", subcore_axis_name="s")
    fn = functools.partial(
        pl.kernel,
        mesh=mesh,
        out_type=jax.ShapeDtypeStruct((rows, dpad), jnp.float32),
        scratch_types=[
            pltpu.VMEM((b,), jnp.int32),
            pltpu.VMEM((b, dpad), jnp.float32),
            pltpu.SemaphoreType.DMA,
        ],
    )(_sc_gather_body)
    return fn(w_pad, idx)


def kernel(latents, W):
    orig_shape = latents.shape
    d = orig_shape[-1]
    x = latents.reshape(-1, d)
    rows = x.shape[0]

    idx, loss, perp = _stage_a(x, W)
    w_pad = jnp.pad(W, ((0, 0), (0, 128 - d)))
    q = _sc_gather(w_pad, idx)
    return (q[:, :d].reshape(orig_shape), loss.reshape(()), perp.reshape(()))


# speedup vs baseline: 1.5014x; 1.0531x over previous
import functools

import jax
import jax.numpy as jnp
from jax import lax
from jax.experimental import pallas as pl
from jax.experimental.pallas import tpu as pltpu
from jax.experimental.pallas import tpu_sc as plsc

_NC = 2
_NS = 16
_NW = _NC * _NS

_BLK = 1024


def _stage_a_body(x_ref, w_ref, idx_ref, loss_ref, perp_ref, acc_ref, cnt_ref,
                  nw_ref, w2_ref):
    i = pl.program_id(0)
    n = pl.num_programs(0)
    blk = x_ref.shape[0] * x_ref.shape[1]
    k = w_ref.shape[0]
    nch = k // 128

    @pl.when(i == 0)
    def _():
        acc_ref[...] = jnp.zeros_like(acc_ref)
        cnt_ref[...] = jnp.zeros_like(cnt_ref)
        w = w_ref[...]
        nw_ref[...] = -(w + w)
        o8 = jnp.ones((8, w.shape[1]), jnp.float32)
        w2_ref[...] = lax.dot_general(w * w, o8, (((1,), (1,)), ((), ())),
                                      preferred_element_type=jnp.float32)

    x = x_ref[0]
    ones8 = jnp.ones((8, x.shape[1]), jnp.float32)
    x2row = lax.dot_general(ones8, x * x, (((1,), (1,)), ((), ())),
                            preferred_element_type=jnp.float32)[0:1]
    w2 = w2_ref[:, 0:1]
    ji = lax.broadcasted_iota(jnp.int32, (128, blk), 0).astype(jnp.float32)

    minval = None
    minidx = None
    for kb in range(nch):
        nwk = nw_ref[kb * 128:(kb + 1) * 128, :]
        mm2k = lax.dot_general(nwk, x, (((1,), (1,)), ((), ())),
                               preferred_element_type=jnp.float32)
        dk = (x2row + w2[kb * 128:(kb + 1) * 128]) + mm2k
        if kb == 0:
            minval = dk
            minidx = ji
        else:
            better = dk < minval
            minval = jnp.where(better, dk, minval)
            minidx = jnp.where(better, ji + float(kb * 128), minidx)

    m = jnp.min(minval, axis=0, keepdims=True)
    cand = jnp.where(minval == m, minidx, jnp.float32(2.0 ** 30))
    idxf = jnp.min(cand, axis=0, keepdims=True)
    idx_ref[...] = idxf.astype(jnp.int32).reshape(idx_ref.shape)

    acc_ref[...] += jnp.sum(m, keepdims=True)
    rowsc = lax.broadcasted_iota(jnp.int32, (k, blk), 0).astype(jnp.float32)
    onehot = jnp.where(rowsc == idxf, 1.0, 0.0).astype(jnp.float32)
    ones = jnp.ones((8, blk), jnp.float32)
    cnt_ref[...] += lax.dot_general(ones, onehot, (((1,), (1,)), ((), ())),
                                    preferred_element_type=jnp.float32)

    @pl.when(i == n - 1)
    def _():
        rows_total = n * blk
        denom = rows_total * x_ref.shape[2]
        loss_ref[...] = acc_ref[...] * (0.25 / denom)
        p = cnt_ref[0:1, :] * (1.0 / rows_total)
        s = jnp.sum(p * jnp.log(p + 1e-10), keepdims=True)
        perp_ref[...] = jnp.exp(-s)


def _stage_a(x, w):
    nb, t, dd = x.shape
    rows = nb * t
    k = w.shape[0]
    grid = rows // _BLK
    per = max(t // _BLK, 1)
    return pl.pallas_call(
        _stage_a_body,
        grid=(grid,),
        in_specs=[
            pl.BlockSpec((1, _BLK, dd), lambda i: (i // per, i % per, 0)),
            pl.BlockSpec((k, dd), lambda i: (0, 0)),
        ],
        out_specs=[
            pl.BlockSpec((_BLK,), lambda i: (i,)),
            pl.BlockSpec((1, 1), lambda i: (0, 0)),
            pl.BlockSpec((1, 1), lambda i: (0, 0)),
        ],
        out_shape=[
            jax.ShapeDtypeStruct((rows,), jnp.int32),
            jax.ShapeDtypeStruct((1, 1), jnp.float32),
            jax.ShapeDtypeStruct((1, 1), jnp.float32),
        ],
        scratch_shapes=[
            pltpu.VMEM((1, 1), jnp.float32),
            pltpu.VMEM((8, k), jnp.float32),
            pltpu.VMEM((k, dd), jnp.float32),
            pltpu.VMEM((k, 8), jnp.float32),
        ],
    )(x, w)


def _sc_gather_body(w_hbm, idx_hbm, q_hbm, idx_v, rows_v, sem):
    cid = lax.axis_index("c")
    sid = lax.axis_index("s")
    wid = sid * _NC + cid
    b = idx_v.shape[0]
    base = wid * b
    pltpu.sync_copy(idx_hbm.at[pl.ds(base, b)], idx_v)
    pltpu.async_copy(w_hbm.at[idx_v], rows_v, sem).wait()
    pltpu.sync_copy(rows_v, q_hbm.at[pl.ds(base, b)])


def _sc_gather(w_pad, idx):
    rows = idx.shape[0]
    k, dpad = w_pad.shape
    b = rows // _NW
    mesh = plsc.VectorSubcoreMesh(core_axis_name="c", subcore_axis_name="s")
    fn = functools.partial(
        pl.kernel,
        mesh=mesh,
        out_type=jax.ShapeDtypeStruct((rows, dpad), jnp.float32),
        scratch_types=[
            pltpu.VMEM((b,), jnp.int32),
            pltpu.VMEM((b, dpad), jnp.float32),
            pltpu.SemaphoreType.DMA,
        ],
    )(_sc_gather_body)
    return fn(w_pad, idx)


def kernel(latents, W):
    orig_shape = latents.shape
    d = orig_shape[-1]

    idx, loss, perp = _stage_a(latents, W)
    w_pad = jnp.pad(W, ((0, 0), (0, 128 - d)))
    q = _sc_gather(w_pad, idx)
    return (q[:, :d].reshape(orig_shape), loss.reshape(()), perp.reshape(()))


# speedup vs baseline: 1.6564x; 1.1032x over previous
import functools

import jax
import jax.numpy as jnp
from jax import lax
from jax.experimental import pallas as pl
from jax.experimental.pallas import tpu as pltpu
from jax.experimental.pallas import tpu_sc as plsc

_NC = 2
_NS = 16
_NW = _NC * _NS

_BLK = 1024


def _stage_a_body(x_ref, w_ref, idx_ref, loss_ref, perp_ref, acc_ref, cnt_ref,
                  nw_ref, w2_ref):
    i = pl.program_id(0)
    n = pl.num_programs(0)
    blk = x_ref.shape[2]
    k = w_ref.shape[0]
    nch = k // 128

    @pl.when(i == 0)
    def _():
        acc_ref[...] = jnp.zeros_like(acc_ref)
        cnt_ref[...] = jnp.zeros_like(cnt_ref)
        w = w_ref[...]
        nw_ref[...] = -(w + w)
        o8 = jnp.ones((8, w.shape[1]), jnp.float32)
        w2_ref[...] = lax.dot_general(w * w, o8, (((1,), (1,)), ((), ())),
                                      preferred_element_type=jnp.float32)

    xt = x_ref[0]
    ones8 = jnp.ones((8, xt.shape[0]), jnp.float32)
    x2row = lax.dot_general(ones8, xt * xt, (((1,), (0,)), ((), ())),
                            preferred_element_type=jnp.float32)[0:1]
    w2 = w2_ref[:, 0:1]
    ji = lax.broadcasted_iota(jnp.int32, (128, blk), 0).astype(jnp.float32)

    minval = None
    minidx = None
    for kb in range(nch):
        nwk = nw_ref[kb * 128:(kb + 1) * 128, :]
        mm2k = lax.dot_general(nwk, xt, (((1,), (0,)), ((), ())),
                               preferred_element_type=jnp.float32)
        dk = (x2row + w2[kb * 128:(kb + 1) * 128]) + mm2k
        if kb == 0:
            minval = dk
            minidx = ji
        else:
            better = dk < minval
            minval = jnp.where(better, dk, minval)
            minidx = jnp.where(better, ji + float(kb * 128), minidx)

    m = jnp.min(minval, axis=0, keepdims=True)
    cand = jnp.where(minval == m, minidx, jnp.float32(2.0 ** 30))
    idxf = jnp.min(cand, axis=0, keepdims=True)
    idx_ref[...] = idxf.astype(jnp.int32).reshape(idx_ref.shape)

    acc_ref[...] += jnp.sum(m, keepdims=True)
    rowsc = lax.broadcasted_iota(jnp.int32, (k, blk), 0).astype(jnp.float32)
    onehot = jnp.where(rowsc == idxf, 1.0, 0.0).astype(jnp.float32)
    ones = jnp.ones((8, blk), jnp.float32)
    cnt_ref[...] += lax.dot_general(ones, onehot, (((1,), (1,)), ((), ())),
                                    preferred_element_type=jnp.float32)

    @pl.when(i == n - 1)
    def _():
        rows_total = n * blk
        denom = rows_total * x_ref.shape[1]
        loss_ref[...] = acc_ref[...] * (0.25 / denom)
        p = cnt_ref[0:1, :] * (1.0 / rows_total)
        s = jnp.sum(p * jnp.log(p + 1e-10), keepdims=True)
        perp_ref[...] = jnp.exp(-s)


def _stage_a(x, w):
    nb, dd, t = x.shape
    rows = nb * t
    k = w.shape[0]
    grid = rows // _BLK
    return pl.pallas_call(
        _stage_a_body,
        grid=(grid,),
        in_specs=[
            pl.BlockSpec((1, dd, _BLK), lambda i: (i, 0, 0)),
            pl.BlockSpec((k, dd), lambda i: (0, 0)),
        ],
        out_specs=[
            pl.BlockSpec((_BLK,), lambda i: (i,)),
            pl.BlockSpec((1, 1), lambda i: (0, 0)),
            pl.BlockSpec((1, 1), lambda i: (0, 0)),
        ],
        out_shape=[
            jax.ShapeDtypeStruct((rows,), jnp.int32),
            jax.ShapeDtypeStruct((1, 1), jnp.float32),
            jax.ShapeDtypeStruct((1, 1), jnp.float32),
        ],
        scratch_shapes=[
            pltpu.VMEM((1, 1), jnp.float32),
            pltpu.VMEM((8, k), jnp.float32),
            pltpu.VMEM((k, dd), jnp.float32),
            pltpu.VMEM((k, 8), jnp.float32),
        ],
    )(x, w)


def _sc_gather_body(w_hbm, idx_hbm, q_hbm, idx_v, rows_v, sem):
    cid = lax.axis_index("c")
    sid = lax.axis_index("s")
    wid = sid * _NC + cid
    b = idx_v.shape[0]
    base = wid * b
    pltpu.sync_copy(idx_hbm.at[pl.ds(base, b)], idx_v)
    pltpu.async_copy(w_hbm.at[idx_v], rows_v, sem).wait()
    pltpu.sync_copy(rows_v, q_hbm.at[pl.ds(base, b)])


def _sc_gather(w_pad, idx):
    rows = idx.shape[0]
    k, dpad = w_pad.shape
    b = rows // _NW
    mesh = plsc.VectorSubcoreMesh(core_axis_name="c", subcore_axis_name="s")
    fn = functools.partial(
        pl.kernel,
        mesh=mesh,
        out_type=jax.ShapeDtypeStruct((rows, dpad), jnp.float32),
        scratch_types=[
            pltpu.VMEM((b,), jnp.int32),
            pltpu.VMEM((b, dpad), jnp.float32),
            pltpu.SemaphoreType.DMA,
        ],
    )(_sc_gather_body)
    return fn(w_pad, idx)


def kernel(latents, W):
    orig_shape = latents.shape
    d = orig_shape[-1]

    xt = jnp.swapaxes(latents, 1, 2)
    idx, loss, perp = _stage_a(xt, W)
    w_pad = jnp.pad(W, ((0, 0), (0, 128 - d)))
    q = _sc_gather(w_pad, idx)
    return (q[:, :d].reshape(orig_shape), loss.reshape(()), perp.reshape(()))
